# Initial kernel scaffold; baseline (speedup 1.0000x reference)
import jax
import jax.numpy as jnp
from jax.experimental import pallas as pl


def kernel(graph, fts, time_steps, Wl0, bl0, Wr0, Wl1, bl1, Wr1, Wc, bc):
    raise NotImplementedError("write your pallas kernel here")


# speedup vs baseline: 9.7947x; 9.7947x over previous
import functools

import jax
import jax.numpy as jnp
from jax import lax
from jax.experimental import pallas as pl
from jax.experimental.pallas import tpu as pltpu
from jax.experimental.pallas import tpu_sc as plsc

N = 10000
E = 320000
D = 128
T = 4
NW = 32
EPW = E // NW
K = 80
NCH = EPW // K
RPT = 632
NP = RPT * 16


def _make_segmean(src_off, dst_off, with_cnt):
    out_type = [jax.ShapeDtypeStruct((2, NP, D), jnp.float32)]
    if with_cnt:
        out_type.append(jax.ShapeDtypeStruct((2, NP, D), jnp.float32))
    mesh = plsc.VectorSubcoreMesh(core_axis_name="c", subcore_axis_name="s")

    @functools.partial(
        pl.kernel, mesh=mesh, out_type=out_type,
        scratch_types=[
            pltpu.VMEM((K,), jnp.int32),
            pltpu.VMEM((K,), jnp.int32),
            pltpu.VMEM((K, D), jnp.float32),
            pltpu.VMEM_SHARED((NP, D), jnp.float32),
            pltpu.SemaphoreType.DMA,
        ],
        compiler_params=pltpu.CompilerParams(needs_layout_passes=False))
    def seg(table, src_arr, dst_arr, z2, ones2, *rest):
        if with_cnt:
            out_part, out_cnt, src_i, dst_i, rows, acc, sem = rest
        else:
            out_part, src_i, dst_i, rows, acc, sem = rest
            out_cnt = None
        c = lax.axis_index("c")
        s = lax.axis_index("s")
        w = c * 16 + s
        rbase = s * RPT
        rem = RPT % K
        estart = w * EPW

        def zero_acc():
            pltpu.sync_copy(z2, rows)
            for q in range(RPT // K):
                pltpu.sync_copy(rows, acc.at[pl.ds(rbase + q * K, K)])
            if rem:
                pltpu.sync_copy(rows.at[pl.ds(0, rem)],
                                acc.at[pl.ds(rbase + RPT - rem, rem)])

        def writeback(dst_hbm):
            for q in range(RPT // K):
                sl = pl.ds(rbase + q * K, K)
                pltpu.sync_copy(acc.at[sl], rows)
                pltpu.sync_copy(rows, dst_hbm.at[c, sl])
            if rem:
                sl = pl.ds(rbase + RPT - rem, rem)
                pltpu.sync_copy(acc.at[sl], rows.at[pl.ds(0, rem)])
                pltpu.sync_copy(rows.at[pl.ds(0, rem)], dst_hbm.at[c, sl])

        zero_acc()
        plsc.subcore_barrier()

        def chunk(i, carry):
            base = estart + i * K
            pltpu.sync_copy(src_arr.at[pl.ds(src_off + base, K)], src_i)
            pltpu.sync_copy(dst_arr.at[pl.ds(dst_off + base, K)], dst_i)
            pltpu.async_copy(table.at[src_i], rows, sem).wait()
            pltpu.sync_copy(rows, acc.at[dst_i], add=True)
            return carry

        lax.fori_loop(0, NCH, chunk, 0)
        plsc.subcore_barrier()
        writeback(out_part)

        if with_cnt:
            zero_acc()
            pltpu.sync_copy(ones2, rows)
            plsc.subcore_barrier()

            def chunk_cnt(i, carry):
                base = estart + i * K
                pltpu.sync_copy(dst_arr.at[pl.ds(dst_off + base, K)], dst_i)
                pltpu.sync_copy(rows, acc.at[dst_i], add=True)
                return carry

            lax.fori_loop(0, NCH, chunk_cnt, 0)
            plsc.subcore_barrier()
            writeback(out_cnt)

    return seg


_B = 1000


def _dense_layer0(parts, cnts, x, WlT, bl, WrT):
    def body(p_ref, n_ref, x_ref, wl_ref, bl_ref, wr_ref, o_ref, c_ref):
        p = p_ref[0] + p_ref[1]
        cnt = (n_ref[0] + n_ref[1])[:, 0:1]
        agg = p / jnp.maximum(cnt, 1.0)
        o = (jnp.dot(agg, wl_ref[...], preferred_element_type=jnp.float32)
             + bl_ref[...]
             + jnp.dot(x_ref[...], wr_ref[...],
                       preferred_element_type=jnp.float32))
        o_ref[...] = jnp.where(o >= 0, o, 0.2 * o)
        c_ref[...] = cnt

    return pl.pallas_call(
        body,
        grid=(N // _B,),
        in_specs=[
            pl.BlockSpec((2, _B, D), lambda i: (0, i, 0)),
            pl.BlockSpec((2, _B, D), lambda i: (0, i, 0)),
            pl.BlockSpec((_B, D), lambda i: (i, 0)),
            pl.BlockSpec((D, D), lambda i: (0, 0)),
            pl.BlockSpec((1, D), lambda i: (0, 0)),
            pl.BlockSpec((D, D), lambda i: (0, 0)),
        ],
        out_specs=[pl.BlockSpec((_B, D), lambda i: (i, 0)),
                   pl.BlockSpec((_B, 1), lambda i: (i, 0))],
        out_shape=[jax.ShapeDtypeStruct((N, D), jnp.float32),
                   jax.ShapeDtypeStruct((N, 1), jnp.float32)],
    )(parts, cnts, x, WlT, bl, WrT)


def _dense_layer_cls(parts, cnt2d, x, WlT, bl, WrT, WcT, bc):
    nc = WcT.shape[1]

    def body(p_ref, c_ref, x_ref, wl_ref, bl_ref, wr_ref, wc_ref, bc_ref,
             z_ref):
        p = p_ref[0] + p_ref[1]
        agg = p / jnp.maximum(c_ref[...], 1.0)
        o = (jnp.dot(agg, wl_ref[...], preferred_element_type=jnp.float32)
             + bl_ref[...]
             + jnp.dot(x_ref[...], wr_ref[...],
                       preferred_element_type=jnp.float32))
        y = jnp.where(o >= 0, o, 0.2 * o)
        z_ref[...] = (jnp.dot(y, wc_ref[...],
                              preferred_element_type=jnp.float32)
                      + bc_ref[...])

    return pl.pallas_call(
        body,
        grid=(N // _B,),
        in_specs=[
            pl.BlockSpec((2, _B, D), lambda i: (0, i, 0)),
            pl.BlockSpec((_B, 1), lambda i: (i, 0)),
            pl.BlockSpec((_B, D), lambda i: (i, 0)),
            pl.BlockSpec((D, D), lambda i: (0, 0)),
            pl.BlockSpec((1, D), lambda i: (0, 0)),
            pl.BlockSpec((D, D), lambda i: (0, 0)),
            pl.BlockSpec((D, nc), lambda i: (0, 0)),
            pl.BlockSpec((1, nc), lambda i: (0, 0)),
        ],
        out_specs=pl.BlockSpec((_B, nc), lambda i: (i, 0)),
        out_shape=jax.ShapeDtypeStruct((N, nc), jnp.float32),
    )(parts, cnt2d, x, WlT, bl, WrT, WcT, bc)


def _log_softmax_nodes(z):
    def body(z_ref, o_ref):
        zz = z_ref[...]
        m = jnp.max(zz, axis=1, keepdims=True)
        e = jnp.exp(zz - m)
        o_ref[...] = zz - m - jnp.log(jnp.sum(e, axis=1, keepdims=True))

    return pl.pallas_call(
        body, out_shape=jax.ShapeDtypeStruct(z.shape, jnp.float32))(z)


def kernel(graph, fts, time_steps, Wl0, bl0, Wr0, Wl1, bl1, Wr1, Wc, bc):
    flat = fts.reshape(T * N, D)
    src_adj = (graph[:, 0, :] + (jnp.arange(T, dtype=jnp.int32) * N)[:, None]
               ).reshape(-1)
    gflat = graph.reshape(-1)
    z2 = jnp.zeros((K, D), jnp.float32)
    ones2 = jnp.ones((K, D), jnp.float32)

    hs = []
    cnt3 = None
    for t in range(T):
        parts, cnts = _make_segmean(t * E, (t * 2 + 1) * E, True)(
            flat, src_adj, gflat, z2, ones2)
        h_t, cnt_t = _dense_layer0(parts, cnts, fts[t], Wl0.T,
                                   bl0.reshape(1, -1), Wr0.T)
        if t == T - 1:
            cnt3 = cnt_t
        hs.append(h_t)

    zs = []
    for t in range(T):
        parts = _make_segmean((T - 1) * 2 * E, ((T - 1) * 2 + 1) * E, False)(
            hs[t], gflat, gflat, z2, ones2)
        if isinstance(parts, (tuple, list)):
            (parts,) = parts
        zs.append(_dense_layer_cls(parts, cnt3, hs[t], Wl1.T,
                                   bl1.reshape(1, -1), Wr1.T, Wc.T,
                                   bc.reshape(1, -1)))

    z = jnp.stack(zs, axis=0)
    return _log_softmax_nodes(z).reshape(N, -1)


# speedup vs baseline: 16.4047x; 1.6749x over previous
import functools

import jax
import jax.numpy as jnp
from jax import lax
from jax.experimental import pallas as pl
from jax.experimental.pallas import tpu as pltpu
from jax.experimental.pallas import tpu_sc as plsc

N = 10000
E = 320000
D = 128
T = 4
NW = 32
EPW = E // NW
K = 128
NCH = EPW // K
KT = EPW - NCH * K
NPAIR = NCH // 2
RPT = 632
NP = RPT * 16


def _make_segmean(src_off, dst_off, with_cnt):
    out_type = [jax.ShapeDtypeStruct((2, NP, D), jnp.float32)]
    if with_cnt:
        out_type.append(jax.ShapeDtypeStruct((2, NP, D), jnp.float32))
    mesh = plsc.VectorSubcoreMesh(core_axis_name="c", subcore_axis_name="s")

    @functools.partial(
        pl.kernel, mesh=mesh, out_type=out_type,
        scratch_types=[
            pltpu.VMEM((K,), jnp.int32),
            pltpu.VMEM((K,), jnp.int32),
            pltpu.VMEM((K,), jnp.int32),
            pltpu.VMEM((K,), jnp.int32),
            pltpu.VMEM((K, D), jnp.float32),
            pltpu.VMEM((K, D), jnp.float32),
            pltpu.VMEM((KT,), jnp.int32),
            pltpu.VMEM((KT,), jnp.int32),
            pltpu.VMEM((KT, D), jnp.float32),
            pltpu.VMEM_SHARED((NP, D), jnp.float32),
            pltpu.SemaphoreType.DMA,
            pltpu.SemaphoreType.DMA,
            pltpu.SemaphoreType.DMA,
            pltpu.SemaphoreType.DMA,
        ],
        compiler_params=pltpu.CompilerParams(needs_layout_passes=False))
    def seg(table, src_arr, dst_arr, z2, ones2, *rest):
        if with_cnt:
            (out_part, out_cnt, src0, dst0, src1, dst1, rows0, rows1,
             srct, dstt, rowst, acc, g0, g1, t0, t1) = rest
        else:
            (out_part, src0, dst0, src1, dst1, rows0, rows1,
             srct, dstt, rowst, acc, g0, g1, t0, t1) = rest
            out_cnt = None
        c = lax.axis_index("c")
        s = lax.axis_index("s")
        w = c * 16 + s
        rbase = s * RPT
        rem = RPT % K
        estart = w * EPW

        def zero_acc():
            pltpu.sync_copy(z2, rows0)
            for q in range(RPT // K):
                pltpu.sync_copy(rows0, acc.at[pl.ds(rbase + q * K, K)])
            if rem:
                pltpu.sync_copy(rows0.at[pl.ds(0, rem)],
                                acc.at[pl.ds(rbase + RPT - rem, rem)])

        def writeback(dst_hbm):
            for q in range(RPT // K):
                sl = pl.ds(rbase + q * K, K)
                pltpu.sync_copy(acc.at[sl], rows0)
                pltpu.sync_copy(rows0, dst_hbm.at[c, sl])
            if rem:
                sl = pl.ds(rbase + RPT - rem, rem)
                pltpu.sync_copy(acc.at[sl], rows0.at[pl.ds(0, rem)])
                pltpu.sync_copy(rows0.at[pl.ds(0, rem)], dst_hbm.at[c, sl])

        zero_acc()
        plsc.subcore_barrier()

        def pair(j, carry):
            base_a = estart + (2 * j) * K
            base_b = base_a + K
            pltpu.sync_copy(src_arr.at[pl.ds(src_off + base_a, K)], src0)
            pltpu.sync_copy(dst_arr.at[pl.ds(dst_off + base_a, K)], dst0)
            ga = pltpu.async_copy(table.at[src0], rows0, g0)
            pltpu.sync_copy(src_arr.at[pl.ds(src_off + base_b, K)], src1)
            pltpu.sync_copy(dst_arr.at[pl.ds(dst_off + base_b, K)], dst1)
            gb = pltpu.async_copy(table.at[src1], rows1, g1)
            ga.wait()
            sa = pltpu.async_copy(rows0, acc.at[dst0], t0, add=True)
            gb.wait()
            sb = pltpu.async_copy(rows1, acc.at[dst1], t1, add=True)
            sa.wait()
            sb.wait()
            return carry

        lax.fori_loop(0, NPAIR, pair, 0)
        if KT:
            base_t = estart + NCH * K
            pltpu.sync_copy(src_arr.at[pl.ds(src_off + base_t, KT)], srct)
            pltpu.sync_copy(dst_arr.at[pl.ds(dst_off + base_t, KT)], dstt)
            pltpu.async_copy(table.at[srct], rowst, g0).wait()
            pltpu.sync_copy(rowst, acc.at[dstt], add=True)
        plsc.subcore_barrier()
        writeback(out_part)

        if with_cnt:
            zero_acc()
            pltpu.sync_copy(ones2, rows0)
            plsc.subcore_barrier()

            def pair_cnt(j, carry):
                base_a = estart + (2 * j) * K
                base_b = base_a + K
                pltpu.sync_copy(dst_arr.at[pl.ds(dst_off + base_a, K)], dst0)
                sa = pltpu.async_copy(rows0, acc.at[dst0], t0, add=True)
                pltpu.sync_copy(dst_arr.at[pl.ds(dst_off + base_b, K)], dst1)
                sb = pltpu.async_copy(rows0, acc.at[dst1], t1, add=True)
                sa.wait()
                sb.wait()
                return carry

            lax.fori_loop(0, NPAIR, pair_cnt, 0)
            if KT:
                base_t = estart + NCH * K
                pltpu.sync_copy(dst_arr.at[pl.ds(dst_off + base_t, KT)], dstt)
                pltpu.sync_copy(rows0.at[pl.ds(0, KT)], acc.at[dstt],
                                add=True)
            plsc.subcore_barrier()
            writeback(out_cnt)

    return seg


_B = 1000


def _dense_layer0(parts, cnts, x, WlT, bl, WrT):
    def body(p_ref, n_ref, x_ref, wl_ref, bl_ref, wr_ref, o_ref, c_ref):
        p = p_ref[0] + p_ref[1]
        cnt = (n_ref[0] + n_ref[1])[:, 0:1]
        agg = p / jnp.maximum(cnt, 1.0)
        o = (jnp.dot(agg, wl_ref[...], preferred_element_type=jnp.float32)
             + bl_ref[...]
             + jnp.dot(x_ref[...], wr_ref[...],
                       preferred_element_type=jnp.float32))
        o_ref[...] = jnp.where(o >= 0, o, 0.2 * o)
        c_ref[...] = cnt

    return pl.pallas_call(
        body,
        grid=(N // _B,),
        in_specs=[
            pl.BlockSpec((2, _B, D), lambda i: (0, i, 0)),
            pl.BlockSpec((2, _B, D), lambda i: (0, i, 0)),
            pl.BlockSpec((_B, D), lambda i: (i, 0)),
            pl.BlockSpec((D, D), lambda i: (0, 0)),
            pl.BlockSpec((1, D), lambda i: (0, 0)),
            pl.BlockSpec((D, D), lambda i: (0, 0)),
        ],
        out_specs=[pl.BlockSpec((_B, D), lambda i: (i, 0)),
                   pl.BlockSpec((_B, 1), lambda i: (i, 0))],
        out_shape=[jax.ShapeDtypeStruct((N, D), jnp.float32),
                   jax.ShapeDtypeStruct((N, 1), jnp.float32)],
    )(parts, cnts, x, WlT, bl, WrT)


def _dense_layer_cls(parts, cnt2d, x, WlT, bl, WrT, WcT, bc):
    nc = WcT.shape[1]

    def body(p_ref, c_ref, x_ref, wl_ref, bl_ref, wr_ref, wc_ref, bc_ref,
             z_ref):
        p = p_ref[0] + p_ref[1]
        agg = p / jnp.maximum(c_ref[...], 1.0)
        o = (jnp.dot(agg, wl_ref[...], preferred_element_type=jnp.float32)
             + bl_ref[...]
             + jnp.dot(x_ref[...], wr_ref[...],
                       preferred_element_type=jnp.float32))
        y = jnp.where(o >= 0, o, 0.2 * o)
        z_ref[...] = (jnp.dot(y, wc_ref[...],
                              preferred_element_type=jnp.float32)
                      + bc_ref[...])

    return pl.pallas_call(
        body,
        grid=(N // _B,),
        in_specs=[
            pl.BlockSpec((2, _B, D), lambda i: (0, i, 0)),
            pl.BlockSpec((_B, 1), lambda i: (i, 0)),
            pl.BlockSpec((_B, D), lambda i: (i, 0)),
            pl.BlockSpec((D, D), lambda i: (0, 0)),
            pl.BlockSpec((1, D), lambda i: (0, 0)),
            pl.BlockSpec((D, D), lambda i: (0, 0)),
            pl.BlockSpec((D, nc), lambda i: (0, 0)),
            pl.BlockSpec((1, nc), lambda i: (0, 0)),
        ],
        out_specs=pl.BlockSpec((_B, nc), lambda i: (i, 0)),
        out_shape=jax.ShapeDtypeStruct((N, nc), jnp.float32),
    )(parts, cnt2d, x, WlT, bl, WrT, WcT, bc)


def _log_softmax_nodes(z):
    def body(z_ref, o_ref):
        zz = z_ref[...]
        m = jnp.max(zz, axis=1, keepdims=True)
        e = jnp.exp(zz - m)
        o_ref[...] = zz - m - jnp.log(jnp.sum(e, axis=1, keepdims=True))

    return pl.pallas_call(
        body, out_shape=jax.ShapeDtypeStruct(z.shape, jnp.float32))(z)


def kernel(graph, fts, time_steps, Wl0, bl0, Wr0, Wl1, bl1, Wr1, Wc, bc):
    flat = fts.reshape(T * N, D)
    src_adj = (graph[:, 0, :] + (jnp.arange(T, dtype=jnp.int32) * N)[:, None]
               ).reshape(-1)
    gflat = graph.reshape(-1)
    z2 = jnp.zeros((K, D), jnp.float32)
    ones2 = jnp.ones((K, D), jnp.float32)

    hs = []
    cnt3 = None
    for t in range(T):
        parts, cnts = _make_segmean(t * E, (t * 2 + 1) * E, True)(
            flat, src_adj, gflat, z2, ones2)
        h_t, cnt_t = _dense_layer0(parts, cnts, fts[t], Wl0.T,
                                   bl0.reshape(1, -1), Wr0.T)
        if t == T - 1:
            cnt3 = cnt_t
        hs.append(h_t)

    zs = []
    for t in range(T):
        parts = _make_segmean((T - 1) * 2 * E, ((T - 1) * 2 + 1) * E, False)(
            hs[t], gflat, gflat, z2, ones2)
        if isinstance(parts, (tuple, list)):
            (parts,) = parts
        zs.append(_dense_layer_cls(parts, cnt3, hs[t], Wl1.T,
                                   bl1.reshape(1, -1), Wr1.T, Wc.T,
                                   bc.reshape(1, -1)))

    z = jnp.stack(zs, axis=0)
    return _log_softmax_nodes(z).reshape(N, -1)
